# Initial kernel scaffold; baseline (speedup 1.0000x reference)
import jax
import jax.numpy as jnp
from jax.experimental import pallas as pl


def kernel(points):
    raise NotImplementedError("write your pallas kernel here")


# speedup vs baseline: 1.6476x; 1.6476x over previous
import functools

import jax
import jax.numpy as jnp
from jax import lax
from jax.experimental import pallas as pl
from jax.experimental.pallas import tpu as pltpu

_N = 16384
_NP = 1024
_K = 32
_R = 128
_C = 128


def _fps_body(px_ref, py_ref, pz_ref, out_ref, dists_ref):
    b = pl.program_id(0)
    px = px_ref[0]
    py = py_ref[0]
    pz = pz_ref[0]
    dists_ref[...] = jnp.full((_R, _C), jnp.inf, jnp.float32)
    out_ref[b, 0] = jnp.int32(0)

    lane_iota = lax.broadcasted_iota(jnp.int32, (1, _C), 1)
    row_iota = lax.broadcasted_iota(jnp.int32, (_R, _C), 0)
    col_iota = lax.broadcasted_iota(jnp.int32, (_R, _C), 1)
    lin_iota = row_iota * _C + col_iota

    def body(i, last):
        r = last // _C
        c = last % _C
        cm = lane_iota == c
        qx = jnp.sum(jnp.where(cm, px_ref[0, pl.ds(r, 1), :], 0.0))
        qy = jnp.sum(jnp.where(cm, py_ref[0, pl.ds(r, 1), :], 0.0))
        qz = jnp.sum(jnp.where(cm, pz_ref[0, pl.ds(r, 1), :], 0.0))
        dx = px - qx
        dy = py - qy
        dz = pz - qz
        d = dx * dx + dy * dy + dz * dz
        dd = jnp.minimum(dists_ref[...], d)
        dists_ref[...] = dd
        m = jnp.max(dd)
        nxt = jnp.min(jnp.where(dd == m, lin_iota, jnp.int32(2**30)))
        out_ref[b, i] = nxt
        return nxt

    lax.fori_loop(1, _NP, body, jnp.int32(0))


def _fps_pallas(points):
    px = points[:, :, 0].reshape(2, _R, _C)
    py = points[:, :, 1].reshape(2, _R, _C)
    pz = points[:, :, 2].reshape(2, _R, _C)
    return pl.pallas_call(
        _fps_body,
        grid=(2,),
        in_specs=[pl.BlockSpec((1, _R, _C), lambda b: (b, 0, 0))] * 3,
        out_specs=pl.BlockSpec(memory_space=pltpu.SMEM),
        out_shape=jax.ShapeDtypeStruct((2, _NP), jnp.int32),
        scratch_shapes=[pltpu.VMEM((_R, _C), jnp.float32)],
    )(px, py, pz)


def _pdist2squared(x, y):
    xx = jnp.sum(x ** 2, axis=1)[:, :, None]
    yy = jnp.sum(y ** 2, axis=1)[:, None, :]
    dist = xx + yy - 2.0 * jnp.einsum('bdn,bdm->bnm', x, y)
    dist = jnp.nan_to_num(dist, nan=0.0)
    return jnp.clip(dist, 0.0, jnp.inf)


def _knn_ind(xyz2, xyz1, k):
    dist = _pdist2squared(jnp.transpose(xyz2, (0, 2, 1)),
                          jnp.transpose(xyz1, (0, 2, 1)))
    dist_t = jnp.transpose(dist, (0, 2, 1))
    _, idx = jax.lax.top_k(-dist_t, k + 1)
    return idx[:, :, 1:]


def kernel(points):
    B, N, _ = points.shape
    xyz_ind = _fps_pallas(points)
    xyz_query = jax.vmap(lambda p, i: p[i])(points, xyz_ind)

    neighbors_mid = _knn_ind(points, xyz_query, _K)
    src_mid = neighbors_mid.reshape(B, -1)
    dst_mid = jnp.repeat(xyz_ind, _K, axis=1)
    d_mid = jax.vmap(lambda p, d, s: p[d] - p[s])(points, dst_mid, src_mid)

    neighbors_out = _knn_ind(xyz_query, xyz_query, _K)
    src_out = neighbors_out.reshape(B, -1)
    dst_out = jnp.tile(jnp.repeat(jnp.arange(_NP, dtype=jnp.int32), _K)[None, :], (B, 1))
    d_out = jax.vmap(lambda p, d, s: p[d] - p[s])(xyz_query, dst_out, src_out)

    return (xyz_query, d_mid, d_out, xyz_ind, neighbors_mid, neighbors_out)
